# baseline (speedup 1.0000x reference)
import jax
import jax.numpy as jnp
from jax import lax
from jax.experimental import pallas as pl
from jax.experimental.pallas import tpu as pltpu
from jax.experimental.pallas import tpu_sc as plsc

NUM_SC = 2
NUM_SUBCORES = 16
CHUNK = 128


def _npad(n):
    return -(-n // (CHUNK * NUM_SUBCORES)) * (CHUNK * NUM_SUBCORES)


def _sc_segment_sum(table, idx4):
    n, d = table.shape
    nw, nchunk, _, _ = idx4.shape
    assert nw == NUM_SC * NUM_SUBCORES
    npad = _npad(n)
    rows_per_tile = npad // NUM_SUBCORES
    n_rcopy = rows_per_tile // CHUNK

    mesh = plsc.VectorSubcoreMesh(core_axis_name="c", subcore_axis_name="s")

    assert nchunk % 2 == 1
    npair = (nchunk - 1) // 2

    @pl.kernel(
        out_type=[jax.ShapeDtypeStruct((NUM_SC * npad, d), jnp.float32)],
        mesh=mesh,
        scratch_types=[
            pltpu.VMEM((2, CHUNK), jnp.int32),
            pltpu.VMEM((2, CHUNK), jnp.int32),
            pltpu.VMEM((CHUNK, d), jnp.float32),
            pltpu.VMEM((CHUNK, d), jnp.float32),
            pltpu.VMEM_SHARED((npad, d), jnp.float32),
            pltpu.SemaphoreType.DMA,
            pltpu.SemaphoreType.DMA,
            pltpu.SemaphoreType.DMA,
            pltpu.SemaphoreType.DMA,
        ],
    )
    def k(table_hbm, idx_hbm, zd_hbm, outp_hbm,
          idx0, idx1, rows0, rows1, acc,
          gsem0, gsem1, ssem0, ssem1):
        c = lax.axis_index("c")
        s = lax.axis_index("s")
        w = c * NUM_SUBCORES + s
        r0 = s * rows_per_tile
        pltpu.sync_copy(zd_hbm, rows0)

        @pl.loop(0, n_rcopy)
        def _(i):
            pltpu.sync_copy(rows0, acc.at[pl.ds(r0 + i * CHUNK, CHUNK)])

        plsc.subcore_barrier()

        half = CHUNK // 2

        def gather(idx, rows, sem, go):
            for lo in (0, half):
                dma = pltpu.make_async_copy(
                    table_hbm.at[idx.at[0, pl.ds(lo, half)]],
                    rows.at[pl.ds(lo, half)], sem)
                dma.start() if go else dma.wait()

        pltpu.sync_copy(idx_hbm.at[w, 0], idx0)
        gather(idx0, rows0, gsem0, True)
        pltpu.sync_copy(idx_hbm.at[w, 1], idx1)

        @pl.loop(0, npair)
        def _(p):
            i0 = 2 * p
            gather(idx0, rows0, gsem0, False)
            s0 = pltpu.async_copy(rows0, acc.at[idx0.at[1]], ssem0, add=True)

            @pl.when(p > 0)
            def _():
                pltpu.make_async_copy(rows1, acc.at[idx1.at[1]], ssem1).wait()
                pltpu.sync_copy(idx_hbm.at[w, i0 + 1], idx1)

            gather(idx1, rows1, gsem1, True)
            gather(idx1, rows1, gsem1, False)
            pltpu.async_copy(rows1, acc.at[idx1.at[1]], ssem1, add=True)
            s0.wait()
            pltpu.sync_copy(idx_hbm.at[w, i0 + 2], idx0)
            gather(idx0, rows0, gsem0, True)

        gather(idx0, rows0, gsem0, False)
        pltpu.make_async_copy(rows1, acc.at[idx1.at[1]], ssem1).wait()
        pltpu.sync_copy(rows0, acc.at[idx0.at[1]], add=True)

        plsc.subcore_barrier()

        @pl.loop(0, n_rcopy)
        def _(i):
            pltpu.sync_copy(acc.at[pl.ds(r0 + i * CHUNK, CHUNK)], rows0)
            pltpu.sync_copy(rows0, outp_hbm.at[pl.ds(c * npad + r0 + i * CHUNK, CHUNK)])

    out = k(table, idx4, jnp.zeros((CHUNK, d), jnp.float32))
    return out[0].reshape(NUM_SC, npad, d)


def _sc_degree(dst3, n):
    d = 128
    nw, nchunk, _ = dst3.shape
    assert nw == NUM_SC * NUM_SUBCORES
    npad = _npad(n)
    rows_per_tile = npad // NUM_SUBCORES
    n_rcopy = rows_per_tile // CHUNK

    mesh = plsc.VectorSubcoreMesh(core_axis_name="c", subcore_axis_name="s")

    assert nchunk % 2 == 1
    npair = (nchunk - 1) // 2

    @pl.kernel(
        out_type=[jax.ShapeDtypeStruct((NUM_SC * npad, d), jnp.float32)],
        mesh=mesh,
        scratch_types=[
            pltpu.VMEM((nchunk, CHUNK), jnp.int32),
            pltpu.VMEM((CHUNK, d), jnp.float32),
            pltpu.VMEM_SHARED((npad, d), jnp.float32),
            pltpu.SemaphoreType.DMA,
            pltpu.SemaphoreType.DMA,
        ],
    )
    def k(dst_hbm, zd_hbm, ones_hbm, outp_hbm, didx, rows, acc, ssem0, ssem1):
        c = lax.axis_index("c")
        s = lax.axis_index("s")
        w = c * NUM_SUBCORES + s
        r0 = s * rows_per_tile
        pltpu.sync_copy(dst_hbm.at[w], didx)
        pltpu.sync_copy(zd_hbm, rows)

        @pl.loop(0, n_rcopy)
        def _(i):
            pltpu.sync_copy(rows, acc.at[pl.ds(r0 + i * CHUNK, CHUNK)])

        pltpu.sync_copy(ones_hbm, rows)
        plsc.subcore_barrier()

        @pl.loop(0, npair)
        def _(p):
            i0 = 2 * p
            s0 = pltpu.async_copy(rows, acc.at[didx.at[i0]], ssem0, add=True)

            @pl.when(p > 0)
            def _():
                pltpu.make_async_copy(rows, acc.at[didx.at[i0 - 1]], ssem1).wait()

            pltpu.async_copy(rows, acc.at[didx.at[i0 + 1]], ssem1, add=True)
            s0.wait()

        pltpu.make_async_copy(rows, acc.at[didx.at[nchunk - 2]], ssem1).wait()
        pltpu.sync_copy(rows, acc.at[didx.at[nchunk - 1]], add=True)

        plsc.subcore_barrier()

        @pl.loop(0, n_rcopy)
        def _(i):
            pltpu.sync_copy(acc.at[pl.ds(r0 + i * CHUNK, CHUNK)], rows)
            pltpu.sync_copy(rows, outp_hbm.at[pl.ds(c * npad + r0 + i * CHUNK, CHUNK)])

    out = k(dst3, jnp.zeros((CHUNK, d), jnp.float32), jnp.ones((CHUNK, d), jnp.float32))
    return out[0].reshape(NUM_SC, npad, d)


def _dot_t(a, w):
    return lax.dot_general(a, w, (((1,), (1,)), ((), ())),
                           preferred_element_type=jnp.float32)


def _tc_mid_a(p1, x, w1l, w1r, bm):
    n, d_in = x.shape
    d_hid = w1l.shape[0]
    grid = (n // bm,)

    def body(p_ref, x_ref, w1l_ref, w1r_ref, g_ref, r1_ref):
        agg = p_ref[0] + p_ref[1]
        g_ref[...] = _dot_t(agg, w1l_ref[...])
        r1_ref[...] = _dot_t(x_ref[...], w1r_ref[...])

    return pl.pallas_call(
        body,
        grid=grid,
        in_specs=[
            pl.BlockSpec((NUM_SC, bm, d_in), lambda i: (0, i, 0)),
            pl.BlockSpec((bm, d_in), lambda i: (i, 0)),
            pl.BlockSpec((d_hid, d_in), lambda i: (0, 0)),
            pl.BlockSpec((d_hid, d_in), lambda i: (0, 0)),
        ],
        out_specs=[
            pl.BlockSpec((bm, d_hid), lambda i: (i, 0)),
            pl.BlockSpec((bm, d_hid), lambda i: (i, 0)),
        ],
        out_shape=[
            jax.ShapeDtypeStruct((n, d_hid), jnp.float32),
            jax.ShapeDtypeStruct((n, d_hid), jnp.float32),
        ],
    )(p1, x, w1l, w1r)


def _tc_mid_b(g, r1, dp, b1, w2l, w2r, bm):
    n, d_hid = g.shape
    d_out = w2l.shape[0]
    grid = (n // bm,)

    def body(g_ref, r1_ref, dp_ref, b1_ref, w2l_ref, w2r_ref, t2_ref, r2_ref):
        deg = dp_ref[0, :, 0:1] + dp_ref[1, :, 0:1]
        dinv = 1.0 / jnp.maximum(deg, 1.0)
        h = g_ref[...] * dinv + b1_ref[...] + r1_ref[...]
        h = jnp.maximum(h, 0.0)
        t2_ref[...] = _dot_t(h, w2l_ref[...])
        r2_ref[...] = _dot_t(h, w2r_ref[...])

    return pl.pallas_call(
        body,
        grid=grid,
        in_specs=[
            pl.BlockSpec((bm, d_hid), lambda i: (i, 0)),
            pl.BlockSpec((bm, d_hid), lambda i: (i, 0)),
            pl.BlockSpec((NUM_SC, bm, 128), lambda i: (0, i, 0)),
            pl.BlockSpec((1, d_hid), lambda i: (0, 0)),
            pl.BlockSpec((d_out, d_hid), lambda i: (0, 0)),
            pl.BlockSpec((d_out, d_hid), lambda i: (0, 0)),
        ],
        out_specs=[
            pl.BlockSpec((bm, d_out), lambda i: (i, 0)),
            pl.BlockSpec((bm, d_out), lambda i: (i, 0)),
        ],
        out_shape=[
            jax.ShapeDtypeStruct((n, d_out), jnp.float32),
            jax.ShapeDtypeStruct((n, d_out), jnp.float32),
        ],
    )(g, r1, dp, b1, w2l, w2r)


def _tc_final(p2, dp, r2, b2, bm):
    n, d_out = r2.shape
    grid = (n // bm,)

    def body(p2_ref, dp_ref, r2_ref, b2_ref, out_ref):
        agg = p2_ref[0] + p2_ref[1]
        deg = dp_ref[0, :, 0:1] + dp_ref[1, :, 0:1]
        logits = agg / jnp.maximum(deg, 1.0) + b2_ref[...] + r2_ref[...]
        m = jnp.max(logits, axis=1, keepdims=True)
        shifted = logits - m
        lse = jnp.log(jnp.sum(jnp.exp(shifted), axis=1, keepdims=True))
        out_ref[...] = shifted - lse

    return pl.pallas_call(
        body,
        grid=grid,
        in_specs=[
            pl.BlockSpec((NUM_SC, bm, d_out), lambda i: (0, i, 0)),
            pl.BlockSpec((NUM_SC, bm, 128), lambda i: (0, i, 0)),
            pl.BlockSpec((bm, d_out), lambda i: (i, 0)),
            pl.BlockSpec((1, d_out), lambda i: (0, 0)),
        ],
        out_specs=pl.BlockSpec((bm, d_out), lambda i: (i, 0)),
        out_shape=jax.ShapeDtypeStruct((n, d_out), jnp.float32),
    )(p2, dp, r2, b2)


def kernel(x, edge_index, W1_l, b1_l, W1_r, W2_l, b2_l, W2_r):
    edge_index = jnp.asarray(edge_index, jnp.int32)
    src = edge_index[0]
    dst = edge_index[1]
    n = x.shape[0]
    e = src.shape[0]
    bm = 1000

    nw = NUM_SC * NUM_SUBCORES
    npad = _npad(n)
    nchunk = -(-e // (nw * CHUNK))
    if nchunk % 2 == 0:
        nchunk += 1
    e_pad = nw * CHUNK * nchunk
    pad = e_pad - e
    pad_ar = jnp.arange(pad, dtype=jnp.int32)
    src_p = jnp.concatenate([src, pad_ar % n])
    dst_p = jnp.concatenate([dst, n + pad_ar % (npad - n)])
    src3 = src_p.reshape(nw, nchunk, CHUNK)
    dst3 = dst_p.reshape(nw, nchunk, CHUNK)
    idx4 = jnp.stack([src3, dst3], axis=2)

    p1 = _sc_segment_sum(x, idx4)
    dp = _sc_degree(dst3, n)
    g, r1 = _tc_mid_a(p1, x, W1_l, W1_r, bm)
    t2, r2 = _tc_mid_b(g, r1, dp, b1_l.reshape(1, -1), W2_l, W2_r, bm)
    p2 = _sc_segment_sum(t2, idx4)
    return _tc_final(p2, dp, r2, b2_l.reshape(1, -1), bm)


# speedup vs baseline: 1.0186x; 1.0186x over previous
import jax
import jax.numpy as jnp
from jax import lax
from jax.experimental import pallas as pl
from jax.experimental.pallas import tpu as pltpu
from jax.experimental.pallas import tpu_sc as plsc

NUM_SC = 2
NUM_SUBCORES = 16
CHUNK = 128


def _npad(n):
    return -(-n // (CHUNK * NUM_SUBCORES)) * (CHUNK * NUM_SUBCORES)


def _sc_segment_sum(table, idx4):
    n, d = table.shape
    nw, nchunk, _, _ = idx4.shape
    assert nw == NUM_SC * NUM_SUBCORES
    npad = _npad(n)
    rows_per_tile = npad // NUM_SUBCORES
    n_rcopy = rows_per_tile // CHUNK

    mesh = plsc.VectorSubcoreMesh(core_axis_name="c", subcore_axis_name="s")

    assert nchunk % 2 == 1
    npair = (nchunk - 1) // 2

    @pl.kernel(
        out_type=[jax.ShapeDtypeStruct((NUM_SC * npad, d), jnp.float32)],
        mesh=mesh,
        scratch_types=[
            pltpu.VMEM((2, CHUNK), jnp.int32),
            pltpu.VMEM((2, CHUNK), jnp.int32),
            pltpu.VMEM((CHUNK, d), jnp.float32),
            pltpu.VMEM((CHUNK, d), jnp.float32),
            pltpu.VMEM_SHARED((npad, d), jnp.float32),
            pltpu.SemaphoreType.DMA,
            pltpu.SemaphoreType.DMA,
            pltpu.SemaphoreType.DMA,
            pltpu.SemaphoreType.DMA,
        ],
    )
    def k(table_hbm, idx_hbm, zd_hbm, outp_hbm,
          idx0, idx1, rows0, rows1, acc,
          gsem0, gsem1, ssem0, ssem1):
        c = lax.axis_index("c")
        s = lax.axis_index("s")
        w = c * NUM_SUBCORES + s
        r0 = s * rows_per_tile
        pltpu.sync_copy(zd_hbm, rows0)

        @pl.loop(0, n_rcopy)
        def _(i):
            pltpu.sync_copy(rows0, acc.at[pl.ds(r0 + i * CHUNK, CHUNK)])

        plsc.subcore_barrier()

        half = CHUNK // 2

        def gather(idx, rows, sem, go):
            for lo in (0, half):
                dma = pltpu.make_async_copy(
                    table_hbm.at[idx.at[0, pl.ds(lo, half)]],
                    rows.at[pl.ds(lo, half)], sem)
                dma.start() if go else dma.wait()

        pltpu.sync_copy(idx_hbm.at[w, 0], idx0)
        gather(idx0, rows0, gsem0, True)
        pltpu.sync_copy(idx_hbm.at[w, 1], idx1)

        @pl.loop(0, npair)
        def _(p):
            i0 = 2 * p
            gather(idx0, rows0, gsem0, False)
            s0 = pltpu.async_copy(rows0, acc.at[idx0.at[1]], ssem0, add=True)

            @pl.when(p > 0)
            def _():
                pltpu.make_async_copy(rows1, acc.at[idx1.at[1]], ssem1).wait()
                pltpu.sync_copy(idx_hbm.at[w, i0 + 1], idx1)

            gather(idx1, rows1, gsem1, True)
            gather(idx1, rows1, gsem1, False)
            pltpu.async_copy(rows1, acc.at[idx1.at[1]], ssem1, add=True)
            s0.wait()
            pltpu.sync_copy(idx_hbm.at[w, i0 + 2], idx0)
            gather(idx0, rows0, gsem0, True)

        gather(idx0, rows0, gsem0, False)
        pltpu.make_async_copy(rows1, acc.at[idx1.at[1]], ssem1).wait()
        pltpu.sync_copy(rows0, acc.at[idx0.at[1]], add=True)

        plsc.subcore_barrier()

        @pl.loop(0, n_rcopy)
        def _(i):
            pltpu.sync_copy(acc.at[pl.ds(r0 + i * CHUNK, CHUNK)], rows0)
            pltpu.sync_copy(rows0, outp_hbm.at[pl.ds(c * npad + r0 + i * CHUNK, CHUNK)])

    out = k(table, idx4, jnp.zeros((CHUNK, d), jnp.float32))
    return out[0].reshape(NUM_SC, npad, d)


def _sc_degree(dst3, n):
    d = 128
    nw, nchunk, _ = dst3.shape
    assert nw == NUM_SC * NUM_SUBCORES
    npad = _npad(n)
    rows_per_tile = npad // NUM_SUBCORES
    n_rcopy = rows_per_tile // CHUNK

    mesh = plsc.VectorSubcoreMesh(core_axis_name="c", subcore_axis_name="s")

    assert nchunk % 2 == 1
    npair = (nchunk - 1) // 2

    @pl.kernel(
        out_type=[jax.ShapeDtypeStruct((NUM_SC * npad, d), jnp.float32)],
        mesh=mesh,
        scratch_types=[
            pltpu.VMEM((nchunk, CHUNK), jnp.int32),
            pltpu.VMEM((CHUNK, d), jnp.float32),
            pltpu.VMEM_SHARED((npad, d), jnp.float32),
            pltpu.SemaphoreType.DMA,
            pltpu.SemaphoreType.DMA,
        ],
    )
    def k(dst_hbm, zd_hbm, ones_hbm, outp_hbm, didx, rows, acc, ssem0, ssem1):
        c = lax.axis_index("c")
        s = lax.axis_index("s")
        w = c * NUM_SUBCORES + s
        r0 = s * rows_per_tile
        pltpu.sync_copy(dst_hbm.at[w], didx)
        pltpu.sync_copy(zd_hbm, rows)

        @pl.loop(0, n_rcopy)
        def _(i):
            pltpu.sync_copy(rows, acc.at[pl.ds(r0 + i * CHUNK, CHUNK)])

        pltpu.sync_copy(ones_hbm, rows)
        plsc.subcore_barrier()

        @pl.loop(0, npair)
        def _(p):
            i0 = 2 * p
            s0 = pltpu.async_copy(rows, acc.at[didx.at[i0]], ssem0, add=True)

            @pl.when(p > 0)
            def _():
                pltpu.make_async_copy(rows, acc.at[didx.at[i0 - 1]], ssem1).wait()

            pltpu.async_copy(rows, acc.at[didx.at[i0 + 1]], ssem1, add=True)
            s0.wait()

        pltpu.make_async_copy(rows, acc.at[didx.at[nchunk - 2]], ssem1).wait()
        pltpu.sync_copy(rows, acc.at[didx.at[nchunk - 1]], add=True)

        plsc.subcore_barrier()

        @pl.loop(0, n_rcopy)
        def _(i):
            pltpu.sync_copy(acc.at[pl.ds(r0 + i * CHUNK, CHUNK)], rows)
            pltpu.sync_copy(rows, outp_hbm.at[pl.ds(c * npad + r0 + i * CHUNK, CHUNK)])

    out = k(dst3, jnp.zeros((CHUNK, d), jnp.float32), jnp.ones((CHUNK, d), jnp.float32))
    return out[0].reshape(NUM_SC, npad, d)


def _dot_t(a, w):
    return lax.dot_general(a, w, (((1,), (1,)), ((), ())),
                           preferred_element_type=jnp.float32)


def _tc_mid(p1, dp, x, w1l, b1, w1r, w2l, w2r, bm):
    n, d_in = x.shape
    d_hid = w1l.shape[0]
    d_out = w2l.shape[0]
    grid = (n // bm,)

    def body(p_ref, dp_ref, x_ref, w1l_ref, b1_ref, w1r_ref, w2l_ref, w2r_ref,
             t2_ref, r2_ref):
        agg = p_ref[0] + p_ref[1]
        deg = dp_ref[0, :, 0:1] + dp_ref[1, :, 0:1]
        mean = agg / jnp.maximum(deg, 1.0)
        h = _dot_t(mean, w1l_ref[...]) + b1_ref[...] + _dot_t(x_ref[...], w1r_ref[...])
        h = jnp.maximum(h, 0.0)
        t2_ref[...] = _dot_t(h, w2l_ref[...])
        r2_ref[...] = _dot_t(h, w2r_ref[...])

    return pl.pallas_call(
        body,
        grid=grid,
        in_specs=[
            pl.BlockSpec((NUM_SC, bm, d_in), lambda i: (0, i, 0)),
            pl.BlockSpec((NUM_SC, bm, 128), lambda i: (0, i, 0)),
            pl.BlockSpec((bm, d_in), lambda i: (i, 0)),
            pl.BlockSpec((d_hid, d_in), lambda i: (0, 0)),
            pl.BlockSpec((1, d_hid), lambda i: (0, 0)),
            pl.BlockSpec((d_hid, d_in), lambda i: (0, 0)),
            pl.BlockSpec((d_out, d_hid), lambda i: (0, 0)),
            pl.BlockSpec((d_out, d_hid), lambda i: (0, 0)),
        ],
        out_specs=[
            pl.BlockSpec((bm, d_out), lambda i: (i, 0)),
            pl.BlockSpec((bm, d_out), lambda i: (i, 0)),
        ],
        out_shape=[
            jax.ShapeDtypeStruct((n, d_out), jnp.float32),
            jax.ShapeDtypeStruct((n, d_out), jnp.float32),
        ],
    )(p1, dp, x, w1l, b1, w1r, w2l, w2r)


def _tc_final(p2, dp, r2, b2, bm):
    n, d_out = r2.shape
    grid = (n // bm,)

    def body(p2_ref, dp_ref, r2_ref, b2_ref, out_ref):
        agg = p2_ref[0] + p2_ref[1]
        deg = dp_ref[0, :, 0:1] + dp_ref[1, :, 0:1]
        logits = agg / jnp.maximum(deg, 1.0) + b2_ref[...] + r2_ref[...]
        m = jnp.max(logits, axis=1, keepdims=True)
        shifted = logits - m
        lse = jnp.log(jnp.sum(jnp.exp(shifted), axis=1, keepdims=True))
        out_ref[...] = shifted - lse

    return pl.pallas_call(
        body,
        grid=grid,
        in_specs=[
            pl.BlockSpec((NUM_SC, bm, d_out), lambda i: (0, i, 0)),
            pl.BlockSpec((NUM_SC, bm, 128), lambda i: (0, i, 0)),
            pl.BlockSpec((bm, d_out), lambda i: (i, 0)),
            pl.BlockSpec((1, d_out), lambda i: (0, 0)),
        ],
        out_specs=pl.BlockSpec((bm, d_out), lambda i: (i, 0)),
        out_shape=jax.ShapeDtypeStruct((n, d_out), jnp.float32),
    )(p2, dp, r2, b2)


def kernel(x, edge_index, W1_l, b1_l, W1_r, W2_l, b2_l, W2_r):
    edge_index = jnp.asarray(edge_index, jnp.int32)
    src = edge_index[0]
    dst = edge_index[1]
    n = x.shape[0]
    e = src.shape[0]
    bm = 1000

    nw = NUM_SC * NUM_SUBCORES
    npad = _npad(n)
    nchunk = -(-e // (nw * CHUNK))
    if nchunk % 2 == 0:
        nchunk += 1
    e_pad = nw * CHUNK * nchunk
    pad = e_pad - e
    pad_ar = jnp.arange(pad, dtype=jnp.int32)
    src_p = jnp.concatenate([src, pad_ar % n])
    dst_p = jnp.concatenate([dst, n + pad_ar % (npad - n)])
    src3 = src_p.reshape(nw, nchunk, CHUNK)
    dst3 = dst_p.reshape(nw, nchunk, CHUNK)
    idx4 = jnp.stack([src3, dst3], axis=2)

    p1 = _sc_segment_sum(x, idx4)
    dp = _sc_degree(dst3, n)
    t2, r2 = _tc_mid(p1, dp, x, W1_l, b1_l.reshape(1, -1), W1_r, W2_l, W2_r, bm)
    p2 = _sc_segment_sum(t2, idx4)
    return _tc_final(p2, dp, r2, b2_l.reshape(1, -1), bm)
